# baseline (speedup 1.0000x reference)
import functools

import jax
import jax.numpy as jnp
from jax import lax
from jax.experimental import pallas as pl
from jax.experimental.pallas import tpu as pltpu
from jax.experimental.pallas import tpu_sc as plsc

N = 10000
E = 320000
D = 128

NC = 2
NS = 16
NW = NC * NS
EPW = E // NW
K = 80
NCHUNK = EPW // K
CH = 624
TAIL = N - NS * CH

_mesh = plsc.VectorSubcoreMesh(core_axis_name="c", subcore_axis_name="s",
                               num_cores=NC, num_subcores=NS)
_sc_params = pltpu.CompilerParams(use_tc_tiling_on_sc=False)


def _zero_acc(zeros_hbm, acc, s):
    pltpu.sync_copy(zeros_hbm.at[pl.ds(0, CH)], acc.at[pl.ds(s * CH, CH)])

    @pl.when(s == 0)
    def _():
        pltpu.sync_copy(zeros_hbm.at[pl.ds(0, TAIL)],
                        acc.at[pl.ds(NS * CH, TAIL)])


def _drain_acc(acc, out_hbm, c, s):
    pltpu.sync_copy(acc.at[pl.ds(s * CH, CH)],
                    out_hbm.at[c, pl.ds(s * CH, CH)])

    @pl.when(s == 0)
    def _():
        pltpu.sync_copy(acc.at[pl.ds(NS * CH, TAIL)],
                        out_hbm.at[c, pl.ds(NS * CH, TAIL)])


DEG_BATCH = 5


def _deg_body(row3_hbm, ones_hbm, zeros_hbm, deg_hbm, ridx_all, ones_v, acc,
              sem_p, sem_s):
    c = lax.axis_index("c")
    s = lax.axis_index("s")
    wid = s * NC + c
    pltpu.async_copy(zeros_hbm.at[pl.ds(0, CH)], acc.at[pl.ds(s * CH, CH)],
                     sem_p)
    pltpu.async_copy(ones_hbm, ones_v, sem_p)
    pltpu.async_copy(row3_hbm.at[wid], ridx_all, sem_p)

    @pl.when(s == 0)
    def _():
        pltpu.async_copy(zeros_hbm.at[pl.ds(0, TAIL)],
                         acc.at[pl.ds(NS * CH, TAIL)], sem_p)
        pltpu.make_async_copy(zeros_hbm.at[pl.ds(0, TAIL)],
                              acc.at[pl.ds(NS * CH, TAIL)], sem_p).wait()

    pltpu.make_async_copy(zeros_hbm.at[pl.ds(0, CH)],
                          acc.at[pl.ds(s * CH, CH)], sem_p).wait()
    pltpu.make_async_copy(ones_hbm, ones_v, sem_p).wait()
    pltpu.make_async_copy(row3_hbm.at[wid], ridx_all, sem_p).wait()
    plsc.subcore_barrier()

    def body(i, carry):
        for j in range(DEG_BATCH):
            pltpu.async_copy(ones_v, acc.at[ridx_all.at[i * DEG_BATCH + j]],
                             sem_s, add=True)
        for j in range(DEG_BATCH):
            pltpu.make_async_copy(
                ones_v, acc.at[ridx_all.at[i * DEG_BATCH + j]], sem_s).wait()
        return carry

    lax.fori_loop(0, NCHUNK // DEG_BATCH, body, 0)
    plsc.subcore_barrier()
    _drain_acc(acc, deg_hbm, c, s)


_deg_kernel = functools.partial(
    pl.kernel,
    out_type=jax.ShapeDtypeStruct((NC, N, 16), jnp.float32),
    mesh=_mesh,
    scratch_types=[
        pltpu.VMEM((NCHUNK, K), jnp.int32),
        pltpu.VMEM((K, 16), jnp.float32),
        pltpu.VMEM_SHARED((N, 16), jnp.float32),
        pltpu.SemaphoreType.DMA,
        pltpu.SemaphoreType.DMA,
    ],
    compiler_params=_sc_params,
)(_deg_body)


def _agg_body(row3_hbm, col3_hbm, g_hbm, zeros_hbm, out_hbm,
              ridx_all, cidx_all, rows_a, rows_b, acc,
              sem_ga, sem_gb, sem_sa, sem_sb, sem_p):
    c = lax.axis_index("c")
    s = lax.axis_index("s")
    wid = s * NC + c
    pltpu.async_copy(zeros_hbm.at[pl.ds(0, CH)], acc.at[pl.ds(s * CH, CH)],
                     sem_p)
    pltpu.async_copy(row3_hbm.at[wid], ridx_all, sem_p)
    pltpu.async_copy(col3_hbm.at[wid], cidx_all, sem_p)

    @pl.when(s == 0)
    def _():
        pltpu.async_copy(zeros_hbm.at[pl.ds(0, TAIL)],
                         acc.at[pl.ds(NS * CH, TAIL)], sem_p)
        pltpu.make_async_copy(zeros_hbm.at[pl.ds(0, TAIL)],
                              acc.at[pl.ds(NS * CH, TAIL)], sem_p).wait()

    pltpu.make_async_copy(zeros_hbm.at[pl.ds(0, CH)],
                          acc.at[pl.ds(s * CH, CH)], sem_p).wait()
    pltpu.make_async_copy(row3_hbm.at[wid], ridx_all, sem_p).wait()
    pltpu.make_async_copy(col3_hbm.at[wid], cidx_all, sem_p).wait()
    plsc.subcore_barrier()

    def gwait(sem, rows, ci):
        pltpu.make_async_copy(g_hbm.at[cidx_all.at[ci]], rows, sem).wait()

    def swait(sem, rows, ci):
        pltpu.make_async_copy(rows, acc.at[ridx_all.at[ci]], sem).wait()

    pltpu.async_copy(g_hbm.at[cidx_all.at[0]], rows_a, sem_ga)
    pltpu.async_copy(g_hbm.at[cidx_all.at[1]], rows_b, sem_gb)

    def body(i, carry):
        ca = 2 * i
        cb = 2 * i + 1
        gwait(sem_ga, rows_a, ca)
        pltpu.async_copy(rows_a, acc.at[ridx_all.at[ca]], sem_sa, add=True)
        gwait(sem_gb, rows_b, cb)
        pltpu.async_copy(rows_b, acc.at[ridx_all.at[cb]], sem_sb, add=True)
        swait(sem_sa, rows_a, ca)
        pltpu.async_copy(g_hbm.at[cidx_all.at[ca + 2]], rows_a, sem_ga)
        swait(sem_sb, rows_b, cb)

        @pl.when(cb + 2 < NCHUNK)
        def _():
            pltpu.async_copy(g_hbm.at[cidx_all.at[cb + 2]], rows_b, sem_gb)

        return carry

    lax.fori_loop(0, (NCHUNK - 1) // 2, body, 0)
    gwait(sem_ga, rows_a, NCHUNK - 1)
    pltpu.async_copy(rows_a, acc.at[ridx_all.at[NCHUNK - 1]], sem_sa, add=True)
    swait(sem_sa, rows_a, NCHUNK - 1)
    plsc.subcore_barrier()
    _drain_acc(acc, out_hbm, c, s)


_agg_kernel = functools.partial(
    pl.kernel,
    out_type=jax.ShapeDtypeStruct((NC, N, D), jnp.float32),
    mesh=_mesh,
    scratch_types=[
        pltpu.VMEM((NCHUNK, K), jnp.int32),
        pltpu.VMEM((NCHUNK, K), jnp.int32),
        pltpu.VMEM((K, D), jnp.float32),
        pltpu.VMEM((K, D), jnp.float32),
        pltpu.VMEM_SHARED((N, D), jnp.float32),
        pltpu.SemaphoreType.DMA,
        pltpu.SemaphoreType.DMA,
        pltpu.SemaphoreType.DMA,
        pltpu.SemaphoreType.DMA,
        pltpu.SemaphoreType.DMA,
    ],
    compiler_params=_sc_params,
)(_agg_body)

BN = 2000


def _dis_from_parts(deg_parts):
    deg = deg_parts[0, :, 0:1] + deg_parts[1, :, 0:1]
    return jnp.where(deg > 0, lax.rsqrt(deg), 0.0)


def _linear_body(x_ref, w_ref, b_ref, deg_ref, g_ref):
    dis = _dis_from_parts(deg_ref[...])
    h = jnp.dot(x_ref[...], w_ref[...].T,
                preferred_element_type=jnp.float32) + b_ref[...]
    g_ref[...] = dis * h


def _finish_body(part_ref, deg_ref, out_ref):
    dis = _dis_from_parts(deg_ref[...])
    out_ref[...] = dis * (part_ref[0] + part_ref[1])


def kernel(x, edge_index, W, b):
    zeros16 = jnp.zeros((CH, 16), jnp.float32)
    ones16 = jnp.ones((K, 16), jnp.float32)
    zerosD = jnp.zeros((CH, D), jnp.float32)

    row3 = edge_index[0].reshape(NW, NCHUNK, K)
    col3 = edge_index[1].reshape(NW, NCHUNK, K)
    deg_parts = _deg_kernel(row3, ones16, zeros16)

    g = pl.pallas_call(
        _linear_body,
        grid=(N // BN,),
        in_specs=[
            pl.BlockSpec((BN, D), lambda i: (i, 0)),
            pl.BlockSpec((D, D), lambda i: (0, 0)),
            pl.BlockSpec((1, D), lambda i: (0, 0)),
            pl.BlockSpec((NC, BN, 16), lambda i: (0, i, 0)),
        ],
        out_specs=pl.BlockSpec((BN, D), lambda i: (i, 0)),
        out_shape=jax.ShapeDtypeStruct((N, D), jnp.float32),
    )(x, W, b.reshape(1, D), deg_parts)

    parts = _agg_kernel(row3, col3, g, zerosD)

    out = pl.pallas_call(
        _finish_body,
        grid=(N // BN,),
        in_specs=[
            pl.BlockSpec((NC, BN, D), lambda i: (0, i, 0)),
            pl.BlockSpec((NC, BN, 16), lambda i: (0, i, 0)),
        ],
        out_specs=pl.BlockSpec((BN, D), lambda i: (i, 0)),
        out_shape=jax.ShapeDtypeStruct((N, D), jnp.float32),
    )(parts, deg_parts)
    return out


# speedup vs baseline: 1.1842x; 1.1842x over previous
import functools

import jax
import jax.numpy as jnp
from jax import lax
from jax.experimental import pallas as pl
from jax.experimental.pallas import tpu as pltpu
from jax.experimental.pallas import tpu_sc as plsc

N = 10000
E = 320000
D = 128

NC = 2
NS = 16
NW = NC * NS
EPW = E // NW
K = 80
NCHUNK = EPW // K
CH = 624
TAIL = N - NS * CH

_mesh = plsc.VectorSubcoreMesh(core_axis_name="c", subcore_axis_name="s",
                               num_cores=NC, num_subcores=NS)
_sc_params = pltpu.CompilerParams(use_tc_tiling_on_sc=False)


def _zero_acc(zeros_hbm, acc, s):
    pltpu.sync_copy(zeros_hbm.at[pl.ds(0, CH)], acc.at[pl.ds(s * CH, CH)])

    @pl.when(s == 0)
    def _():
        pltpu.sync_copy(zeros_hbm.at[pl.ds(0, TAIL)],
                        acc.at[pl.ds(NS * CH, TAIL)])


def _drain_acc(acc, out_hbm, c, s):
    pltpu.sync_copy(acc.at[pl.ds(s * CH, CH)],
                    out_hbm.at[c, pl.ds(s * CH, CH)])

    @pl.when(s == 0)
    def _():
        pltpu.sync_copy(acc.at[pl.ds(NS * CH, TAIL)],
                        out_hbm.at[c, pl.ds(NS * CH, TAIL)])


DEG_BATCH = 5


def _deg_body(row3_hbm, ones_hbm, zeros_hbm, deg_hbm, ridx_all, ones_v, acc,
              sem_p, sem_s):
    c = lax.axis_index("c")
    s = lax.axis_index("s")
    wid = s * NC + c
    pltpu.async_copy(zeros_hbm.at[pl.ds(0, CH)], acc.at[pl.ds(s * CH, CH)],
                     sem_p)
    pltpu.async_copy(ones_hbm, ones_v, sem_p)
    pltpu.async_copy(row3_hbm.at[wid], ridx_all, sem_p)

    @pl.when(s == 0)
    def _():
        pltpu.async_copy(zeros_hbm.at[pl.ds(0, TAIL)],
                         acc.at[pl.ds(NS * CH, TAIL)], sem_p)
        pltpu.make_async_copy(zeros_hbm.at[pl.ds(0, TAIL)],
                              acc.at[pl.ds(NS * CH, TAIL)], sem_p).wait()

    pltpu.make_async_copy(zeros_hbm.at[pl.ds(0, CH)],
                          acc.at[pl.ds(s * CH, CH)], sem_p).wait()
    pltpu.make_async_copy(ones_hbm, ones_v, sem_p).wait()
    pltpu.make_async_copy(row3_hbm.at[wid], ridx_all, sem_p).wait()
    plsc.subcore_barrier()

    def body(i, carry):
        for j in range(DEG_BATCH):
            pltpu.async_copy(ones_v, acc.at[ridx_all.at[i * DEG_BATCH + j]],
                             sem_s, add=True)
        for j in range(DEG_BATCH):
            pltpu.make_async_copy(
                ones_v, acc.at[ridx_all.at[i * DEG_BATCH + j]], sem_s).wait()
        return carry

    lax.fori_loop(0, NCHUNK // DEG_BATCH, body, 0)
    plsc.subcore_barrier()
    _drain_acc(acc, deg_hbm, c, s)


_deg_kernel = functools.partial(
    pl.kernel,
    out_type=jax.ShapeDtypeStruct((NC, N, 16), jnp.float32),
    mesh=_mesh,
    scratch_types=[
        pltpu.VMEM((NCHUNK, K), jnp.int32),
        pltpu.VMEM((K, 16), jnp.float32),
        pltpu.VMEM_SHARED((N, 16), jnp.float32),
        pltpu.SemaphoreType.DMA,
        pltpu.SemaphoreType.DMA,
    ],
    compiler_params=_sc_params,
)(_deg_body)


def _agg_body(row3_hbm, col3_hbm, g_hbm, zeros_hbm, out_hbm,
              ridx_all, cidx_all, rows_a, rows_b, acc,
              sem_ga, sem_gb, sem_p):
    c = lax.axis_index("c")
    s = lax.axis_index("s")
    wid = s * NC + c
    pltpu.async_copy(zeros_hbm.at[pl.ds(0, CH)], acc.at[pl.ds(s * CH, CH)],
                     sem_p)
    pltpu.async_copy(row3_hbm.at[wid], ridx_all, sem_p)
    pltpu.async_copy(col3_hbm.at[wid], cidx_all, sem_p)

    @pl.when(s == 0)
    def _():
        pltpu.async_copy(zeros_hbm.at[pl.ds(0, TAIL)],
                         acc.at[pl.ds(NS * CH, TAIL)], sem_p)
        pltpu.make_async_copy(zeros_hbm.at[pl.ds(0, TAIL)],
                              acc.at[pl.ds(NS * CH, TAIL)], sem_p).wait()

    pltpu.make_async_copy(zeros_hbm.at[pl.ds(0, CH)],
                          acc.at[pl.ds(s * CH, CH)], sem_p).wait()
    pltpu.make_async_copy(row3_hbm.at[wid], ridx_all, sem_p).wait()
    pltpu.make_async_copy(col3_hbm.at[wid], cidx_all, sem_p).wait()
    plsc.subcore_barrier()

    pltpu.async_copy(g_hbm.at[cidx_all.at[0]], rows_a, sem_ga)

    def body(i, carry):
        pltpu.async_copy(g_hbm.at[cidx_all.at[2 * i + 1]], rows_b, sem_gb)
        pltpu.make_async_copy(g_hbm.at[cidx_all.at[2 * i]], rows_a, sem_ga).wait()
        pltpu.sync_copy(rows_a, acc.at[ridx_all.at[2 * i]], add=True)
        pltpu.async_copy(g_hbm.at[cidx_all.at[2 * i + 2]], rows_a, sem_ga)
        pltpu.make_async_copy(g_hbm.at[cidx_all.at[2 * i + 1]], rows_b, sem_gb).wait()
        pltpu.sync_copy(rows_b, acc.at[ridx_all.at[2 * i + 1]], add=True)
        return carry

    lax.fori_loop(0, (NCHUNK - 1) // 2, body, 0)
    pltpu.make_async_copy(g_hbm.at[cidx_all.at[NCHUNK - 1]], rows_a, sem_ga).wait()
    pltpu.sync_copy(rows_a, acc.at[ridx_all.at[NCHUNK - 1]], add=True)
    plsc.subcore_barrier()
    _drain_acc(acc, out_hbm, c, s)


_agg_kernel = functools.partial(
    pl.kernel,
    out_type=jax.ShapeDtypeStruct((NC, N, D), jnp.float32),
    mesh=_mesh,
    scratch_types=[
        pltpu.VMEM((NCHUNK, K), jnp.int32),
        pltpu.VMEM((NCHUNK, K), jnp.int32),
        pltpu.VMEM((K, D), jnp.float32),
        pltpu.VMEM((K, D), jnp.float32),
        pltpu.VMEM_SHARED((N, D), jnp.float32),
        pltpu.SemaphoreType.DMA,
        pltpu.SemaphoreType.DMA,
        pltpu.SemaphoreType.DMA,
    ],
    compiler_params=_sc_params,
)(_agg_body)

BN = 2000


def _dis_from_parts(deg_parts):
    deg = deg_parts[0, :, 0:1] + deg_parts[1, :, 0:1]
    return jnp.where(deg > 0, lax.rsqrt(deg), 0.0)


def _linear_body(x_ref, w_ref, b_ref, deg_ref, g_ref):
    dis = _dis_from_parts(deg_ref[...])
    h = jnp.dot(x_ref[...], w_ref[...].T,
                preferred_element_type=jnp.float32) + b_ref[...]
    g_ref[...] = dis * h


def _finish_body(part_ref, deg_ref, out_ref):
    dis = _dis_from_parts(deg_ref[...])
    out_ref[...] = dis * (part_ref[0] + part_ref[1])


def kernel(x, edge_index, W, b):
    zeros16 = jnp.zeros((CH, 16), jnp.float32)
    ones16 = jnp.ones((K, 16), jnp.float32)
    zerosD = jnp.zeros((CH, D), jnp.float32)

    row3 = edge_index[0].reshape(NW, NCHUNK, K)
    col3 = edge_index[1].reshape(NW, NCHUNK, K)
    deg_parts = _deg_kernel(row3, ones16, zeros16)

    g = pl.pallas_call(
        _linear_body,
        grid=(N // BN,),
        in_specs=[
            pl.BlockSpec((BN, D), lambda i: (i, 0)),
            pl.BlockSpec((D, D), lambda i: (0, 0)),
            pl.BlockSpec((1, D), lambda i: (0, 0)),
            pl.BlockSpec((NC, BN, 16), lambda i: (0, i, 0)),
        ],
        out_specs=pl.BlockSpec((BN, D), lambda i: (i, 0)),
        out_shape=jax.ShapeDtypeStruct((N, D), jnp.float32),
    )(x, W, b.reshape(1, D), deg_parts)

    parts = _agg_kernel(row3, col3, g, zerosD)

    out = pl.pallas_call(
        _finish_body,
        grid=(N // BN,),
        in_specs=[
            pl.BlockSpec((NC, BN, D), lambda i: (0, i, 0)),
            pl.BlockSpec((NC, BN, 16), lambda i: (0, i, 0)),
        ],
        out_specs=pl.BlockSpec((BN, D), lambda i: (i, 0)),
        out_shape=jax.ShapeDtypeStruct((N, D), jnp.float32),
    )(parts, deg_parts)
    return out


# speedup vs baseline: 1.2173x; 1.0280x over previous
import functools

import jax
import jax.numpy as jnp
from jax import lax
from jax.experimental import pallas as pl
from jax.experimental.pallas import tpu as pltpu
from jax.experimental.pallas import tpu_sc as plsc

N = 10000
E = 320000
D = 128

NC = 2
NS = 16
NW = NC * NS
EPW = E // NW
K = 100
NCHUNK = EPW // K
CH = 624
TAIL = N - NS * CH

_mesh = plsc.VectorSubcoreMesh(core_axis_name="c", subcore_axis_name="s",
                               num_cores=NC, num_subcores=NS)
_sc_params = pltpu.CompilerParams(use_tc_tiling_on_sc=False)


def _zero_acc(zeros_hbm, acc, s):
    pltpu.sync_copy(zeros_hbm.at[pl.ds(0, CH)], acc.at[pl.ds(s * CH, CH)])

    @pl.when(s == 0)
    def _():
        pltpu.sync_copy(zeros_hbm.at[pl.ds(0, TAIL)],
                        acc.at[pl.ds(NS * CH, TAIL)])


def _drain_acc(acc, out_hbm, c, s):
    pltpu.sync_copy(acc.at[pl.ds(s * CH, CH)],
                    out_hbm.at[c, pl.ds(s * CH, CH)])

    @pl.when(s == 0)
    def _():
        pltpu.sync_copy(acc.at[pl.ds(NS * CH, TAIL)],
                        out_hbm.at[c, pl.ds(NS * CH, TAIL)])


DEG_BATCH = 5


def _deg_body(row3_hbm, ones_hbm, zeros_hbm, deg_hbm, ridx_all, ones_v, acc,
              sem_p, sem_s):
    c = lax.axis_index("c")
    s = lax.axis_index("s")
    wid = s * NC + c
    pltpu.async_copy(zeros_hbm.at[pl.ds(0, CH)], acc.at[pl.ds(s * CH, CH)],
                     sem_p)
    pltpu.async_copy(ones_hbm, ones_v, sem_p)
    pltpu.async_copy(row3_hbm.at[wid], ridx_all, sem_p)

    @pl.when(s == 0)
    def _():
        pltpu.async_copy(zeros_hbm.at[pl.ds(0, TAIL)],
                         acc.at[pl.ds(NS * CH, TAIL)], sem_p)
        pltpu.make_async_copy(zeros_hbm.at[pl.ds(0, TAIL)],
                              acc.at[pl.ds(NS * CH, TAIL)], sem_p).wait()

    pltpu.make_async_copy(zeros_hbm.at[pl.ds(0, CH)],
                          acc.at[pl.ds(s * CH, CH)], sem_p).wait()
    pltpu.make_async_copy(ones_hbm, ones_v, sem_p).wait()
    pltpu.make_async_copy(row3_hbm.at[wid], ridx_all, sem_p).wait()
    plsc.subcore_barrier()

    def body(i, carry):
        for j in range(DEG_BATCH):
            pltpu.async_copy(ones_v, acc.at[ridx_all.at[i * DEG_BATCH + j]],
                             sem_s, add=True)
        for j in range(DEG_BATCH):
            pltpu.make_async_copy(
                ones_v, acc.at[ridx_all.at[i * DEG_BATCH + j]], sem_s).wait()
        return carry

    lax.fori_loop(0, NCHUNK // DEG_BATCH, body, 0)
    plsc.subcore_barrier()
    _drain_acc(acc, deg_hbm, c, s)


_deg_kernel = functools.partial(
    pl.kernel,
    out_type=jax.ShapeDtypeStruct((NC, N, 16), jnp.float32),
    mesh=_mesh,
    scratch_types=[
        pltpu.VMEM((NCHUNK, K), jnp.int32),
        pltpu.VMEM((K, 16), jnp.float32),
        pltpu.VMEM_SHARED((N, 16), jnp.float32),
        pltpu.SemaphoreType.DMA,
        pltpu.SemaphoreType.DMA,
    ],
    compiler_params=_sc_params,
)(_deg_body)


def _agg_body(row3_hbm, col3_hbm, g_hbm, zeros_hbm, out_hbm,
              ridx_all, cidx_all, rows_a, rows_b, acc,
              sem_ga, sem_gb, sem_p):
    c = lax.axis_index("c")
    s = lax.axis_index("s")
    wid = s * NC + c
    pltpu.async_copy(zeros_hbm.at[pl.ds(0, CH)], acc.at[pl.ds(s * CH, CH)],
                     sem_p)
    pltpu.async_copy(row3_hbm.at[wid], ridx_all, sem_p)
    pltpu.async_copy(col3_hbm.at[wid], cidx_all, sem_p)

    @pl.when(s == 0)
    def _():
        pltpu.async_copy(zeros_hbm.at[pl.ds(0, TAIL)],
                         acc.at[pl.ds(NS * CH, TAIL)], sem_p)
        pltpu.make_async_copy(zeros_hbm.at[pl.ds(0, TAIL)],
                              acc.at[pl.ds(NS * CH, TAIL)], sem_p).wait()

    pltpu.make_async_copy(zeros_hbm.at[pl.ds(0, CH)],
                          acc.at[pl.ds(s * CH, CH)], sem_p).wait()
    pltpu.make_async_copy(row3_hbm.at[wid], ridx_all, sem_p).wait()
    pltpu.make_async_copy(col3_hbm.at[wid], cidx_all, sem_p).wait()
    plsc.subcore_barrier()

    pltpu.async_copy(g_hbm.at[cidx_all.at[0]], rows_a, sem_ga)

    def body(i, carry):
        pltpu.async_copy(g_hbm.at[cidx_all.at[2 * i + 1]], rows_b, sem_gb)
        pltpu.make_async_copy(g_hbm.at[cidx_all.at[2 * i]], rows_a, sem_ga).wait()
        pltpu.sync_copy(rows_a, acc.at[ridx_all.at[2 * i]], add=True)
        pltpu.async_copy(g_hbm.at[cidx_all.at[2 * i + 2]], rows_a, sem_ga)
        pltpu.make_async_copy(g_hbm.at[cidx_all.at[2 * i + 1]], rows_b, sem_gb).wait()
        pltpu.sync_copy(rows_b, acc.at[ridx_all.at[2 * i + 1]], add=True)
        return carry

    lax.fori_loop(0, NCHUNK // 2 - 1, body, 0)
    ca = NCHUNK - 2
    cb = NCHUNK - 1
    pltpu.async_copy(g_hbm.at[cidx_all.at[cb]], rows_b, sem_gb)
    pltpu.make_async_copy(g_hbm.at[cidx_all.at[ca]], rows_a, sem_ga).wait()
    pltpu.sync_copy(rows_a, acc.at[ridx_all.at[ca]], add=True)
    pltpu.make_async_copy(g_hbm.at[cidx_all.at[cb]], rows_b, sem_gb).wait()
    pltpu.sync_copy(rows_b, acc.at[ridx_all.at[cb]], add=True)
    plsc.subcore_barrier()
    _drain_acc(acc, out_hbm, c, s)


_agg_kernel = functools.partial(
    pl.kernel,
    out_type=jax.ShapeDtypeStruct((NC, N, D), jnp.float32),
    mesh=_mesh,
    scratch_types=[
        pltpu.VMEM((NCHUNK, K), jnp.int32),
        pltpu.VMEM((NCHUNK, K), jnp.int32),
        pltpu.VMEM((K, D), jnp.float32),
        pltpu.VMEM((K, D), jnp.float32),
        pltpu.VMEM_SHARED((N, D), jnp.float32),
        pltpu.SemaphoreType.DMA,
        pltpu.SemaphoreType.DMA,
        pltpu.SemaphoreType.DMA,
    ],
    compiler_params=_sc_params,
)(_agg_body)

BN = 2000


def _dis_from_parts(deg_parts):
    deg = deg_parts[0, :, 0:1] + deg_parts[1, :, 0:1]
    return jnp.where(deg > 0, lax.rsqrt(deg), 0.0)


def _linear_body(x_ref, w_ref, b_ref, deg_ref, g_ref):
    dis = _dis_from_parts(deg_ref[...])
    h = jnp.dot(x_ref[...], w_ref[...].T,
                preferred_element_type=jnp.float32) + b_ref[...]
    g_ref[...] = dis * h


def _finish_body(part_ref, deg_ref, out_ref):
    dis = _dis_from_parts(deg_ref[...])
    out_ref[...] = dis * (part_ref[0] + part_ref[1])


def kernel(x, edge_index, W, b):
    zeros16 = jnp.zeros((CH, 16), jnp.float32)
    ones16 = jnp.ones((K, 16), jnp.float32)
    zerosD = jnp.zeros((CH, D), jnp.float32)

    row3 = edge_index[0].reshape(NW, NCHUNK, K)
    col3 = edge_index[1].reshape(NW, NCHUNK, K)
    deg_parts = _deg_kernel(row3, ones16, zeros16)

    g = pl.pallas_call(
        _linear_body,
        grid=(N // BN,),
        in_specs=[
            pl.BlockSpec((BN, D), lambda i: (i, 0)),
            pl.BlockSpec((D, D), lambda i: (0, 0)),
            pl.BlockSpec((1, D), lambda i: (0, 0)),
            pl.BlockSpec((NC, BN, 16), lambda i: (0, i, 0)),
        ],
        out_specs=pl.BlockSpec((BN, D), lambda i: (i, 0)),
        out_shape=jax.ShapeDtypeStruct((N, D), jnp.float32),
    )(x, W, b.reshape(1, D), deg_parts)

    parts = _agg_kernel(row3, col3, g, zerosD)

    out = pl.pallas_call(
        _finish_body,
        grid=(N // BN,),
        in_specs=[
            pl.BlockSpec((NC, BN, D), lambda i: (0, i, 0)),
            pl.BlockSpec((NC, BN, 16), lambda i: (0, i, 0)),
        ],
        out_specs=pl.BlockSpec((BN, D), lambda i: (i, 0)),
        out_shape=jax.ShapeDtypeStruct((N, D), jnp.float32),
    )(parts, deg_parts)
    return out
